# baseline (speedup 1.0000x reference)
import functools

import jax
import jax.numpy as jnp
from jax import lax
from jax.experimental import pallas as pl
from jax.experimental.pallas import tpu as pltpu
from jax.experimental.pallas import tpu_sc as plsc

_NC = 2
_NS = 16
_NW = _NC * _NS
_L = 16

_D = 128
_CH = 64
_RCH = 40


def _proj_body(x_ref, wl_ref, bl_ref, wr_ref, br_ref, xl_ref, xr_ref):
    xb = x_ref[...]
    xl_ref[...] = jnp.dot(xb, wl_ref[...], preferred_element_type=jnp.float32) + bl_ref[...]
    xr_ref[...] = jnp.dot(xb, wr_ref[...], preferred_element_type=jnp.float32) + br_ref[...]


def _project(x, W_l, b_l, W_r, b_r):
    n, d = x.shape
    bn = 2000
    return pl.pallas_call(
        _proj_body,
        grid=(n // bn,),
        in_specs=[
            pl.BlockSpec((bn, d), lambda i: (i, 0)),
            pl.BlockSpec((d, d), lambda i: (0, 0)),
            pl.BlockSpec((1, d), lambda i: (0, 0)),
            pl.BlockSpec((d, d), lambda i: (0, 0)),
            pl.BlockSpec((1, d), lambda i: (0, 0)),
        ],
        out_specs=[
            pl.BlockSpec((bn, d), lambda i: (i, 0)),
            pl.BlockSpec((bn, d), lambda i: (i, 0)),
        ],
        out_shape=[
            jax.ShapeDtypeStruct((n, d), jnp.float32),
            jax.ShapeDtypeStruct((n, d), jnp.float32),
        ],
    )(x, W_l, b_l.reshape(1, d), W_r, b_r.reshape(1, d))


def _finalize_body(acc_ref, den_ref, bias_ref, out_ref):
    d = jnp.sum(den_ref[...], axis=0)
    a = acc_ref[0] + acc_ref[1]
    safe = jnp.where(d > 0, d, 1.0)
    out_ref[...] = a / safe[:, None] + bias_ref[...]


def _finalize(acc, den, bias):
    n = acc.shape[1]
    return pl.pallas_call(
        _finalize_body,
        out_shape=jax.ShapeDtypeStruct((n, _D), jnp.float32),
    )(acc, den, bias.reshape(1, _D))


def _sc_edge_pass(x_l, x_r, src, dst, att):
    n = x_l.shape[0]
    e = src.shape[0]
    assert e % _NW == 0
    per_tile = e // _NW
    n_main = per_tile // _CH
    tail = per_tile - n_main * _CH
    assert tail % _L == 0
    n_chunks = n_main + (1 if tail else 0)
    tail_off = per_tile - _CH
    live_from_group = (_CH - tail) // _L if tail else 0
    assert n_chunks >= 7 and (n_chunks - 4) % 3 == 0
    assert n % _RCH == 0 and _RCH <= _CH
    nzc = n // _RCH
    zk = (nzc + _NS - 1) // _NS

    mesh = plsc.VectorSubcoreMesh(core_axis_name="c", subcore_axis_name="s",
                                  num_cores=_NC, num_subcores=_NS)

    @functools.partial(
        pl.kernel,
        out_type=[
            jax.ShapeDtypeStruct((_NC, n, _D), jnp.float32),
            jax.ShapeDtypeStruct((_NC * n,), jnp.float32),
        ],
        mesh=mesh,
        compiler_params=pltpu.CompilerParams(needs_layout_passes=False),
        scratch_types=[
            pltpu.VMEM((2, _CH), jnp.int32),
            pltpu.VMEM((2, _CH), jnp.int32),
            pltpu.VMEM((2, _CH), jnp.int32),
            pltpu.VMEM((_CH, _D), jnp.float32),
            pltpu.VMEM((_CH, _D), jnp.float32),
            pltpu.VMEM((_CH, _D), jnp.float32),
            pltpu.VMEM((_CH, _D), jnp.float32),
            pltpu.VMEM((_CH, _D), jnp.float32),
            pltpu.VMEM((_CH, _D), jnp.float32),
            pltpu.VMEM((_CH,), jnp.float32),
            pltpu.VMEM((_CH,), jnp.float32),
            pltpu.VMEM((_CH,), jnp.float32),
            pltpu.VMEM((_D,), jnp.float32),
            pltpu.VMEM_SHARED((n, _D), jnp.float32),
            pltpu.VMEM_SHARED((n,), jnp.float32),
            pltpu.SemaphoreType.DMA,
            pltpu.SemaphoreType.DMA,
            pltpu.SemaphoreType.DMA,
            pltpu.SemaphoreType.DMA,
            pltpu.SemaphoreType.DMA,
            pltpu.SemaphoreType.DMA,
        ],
    )
    def sc_kernel(xl_hbm, xr_hbm, src_hbm, dst_hbm, att_hbm, acc_hbm, den_hbm,
                  idx0, idx1, idx2, xl0, xr0, xl1, xr1, xl2, xr2, p0, p1, p2,
                  att_v, acc_sh, den_sh,
                  gsem0, gsem1, gsem2, ssem0, ssem1, ssem2):
        sets = [
            (idx0, xl0, xr0, p0, gsem0, ssem0),
            (idx1, xl1, xr1, p1, gsem1, ssem1),
            (idx2, xl2, xr2, p2, gsem2, ssem2),
        ]
        xlA, pA = xl0, p0
        cid = lax.axis_index("c")
        sid = lax.axis_index("s")
        wid = cid * _NS + sid
        base = wid * per_tile

        pltpu.sync_copy(att_hbm, att_v)

        z16 = jnp.zeros((_L,), jnp.float32)

        for q in range(_CH // _L):
            pA[pl.ds(q * _L, _L)] = z16

        def zrow(i, carry):
            xlA[i // (_D // _L), pl.ds((i % (_D // _L)) * _L, _L)] = z16
            return carry

        lax.fori_loop(0, _CH * (_D // _L), zrow, 0)

        def zacc(k, carry):
            c = sid + k * _NS

            @pl.when(c < nzc)
            def _():
                pltpu.sync_copy(xlA.at[pl.ds(0, _RCH)],
                                acc_sh.at[pl.ds(c * _RCH, _RCH)])
                pltpu.sync_copy(pA.at[pl.ds(0, _RCH)],
                                den_sh.at[pl.ds(c * _RCH, _RCH)])

            return carry

        lax.fori_loop(0, zk, zacc, 0)
        plsc.subcore_barrier()

        att_regs = [att_v[pl.ds(j * _L, _L)] for j in range(_D // _L)]
        lane = lax.broadcasted_iota(jnp.int32, (_L,), 0)
        lane16 = lane * _L

        def issue(ci, S):
            idx_v, xl_v, xr_v, _, gsem, _ = S
            off = base + jnp.minimum(ci * _CH, tail_off)
            pltpu.sync_copy(src_hbm.at[pl.ds(off, _CH)], idx_v.at[0])
            pltpu.sync_copy(dst_hbm.at[pl.ds(off, _CH)], idx_v.at[1])
            pltpu.async_copy(xl_hbm.at[idx_v.at[0]], xl_v, gsem)
            pltpu.async_copy(xr_hbm.at[idx_v.at[1]], xr_v, gsem)

        def wait_g(S):
            idx_v, xl_v, xr_v, _, gsem, _ = S
            pltpu.make_async_copy(xl_hbm.at[idx_v.at[0]], xl_v, gsem).wait()
            pltpu.make_async_copy(xr_hbm.at[idx_v.at[1]], xr_v, gsem).wait()

        def scatter(S):
            idx_v, xl_v, _, p_v, _, ssem = S
            pltpu.async_copy(xl_v, acc_sh.at[idx_v.at[1]], ssem, add=True)
            pltpu.async_copy(p_v, den_sh.at[idx_v.at[1]], ssem, add=True)

        def wait_s(S):
            idx_v, xl_v, _, p_v, _, ssem = S
            pltpu.make_async_copy(xl_v, acc_sh.at[idx_v.at[1]], ssem).wait()
            pltpu.make_async_copy(p_v, den_sh.at[idx_v.at[1]], ssem).wait()

        def compute(c, S):
            idx_v, xl_v, xr_v, p_v, _, _ = S
            def group(g, carry):
                for q in range(_L):
                    k = g * _L + q
                    acc = jnp.zeros((_L,), jnp.float32)
                    for j in range(_D // _L):
                        a = xl_v[k, pl.ds(j * _L, _L)]
                        b = xr_v[k, pl.ds(j * _L, _L)]
                        v = a + b
                        acc = acc + jnp.maximum(v, 0.2 * v) * att_regs[j]
                    xr_v[k, pl.ds(0, _L)] = acc
                rowv = g * _L + lane
                tot = jnp.zeros((_L,), jnp.float32)
                for l in range(_L):
                    tot = tot + plsc.load_gather(
                        xr_v, [rowv, jnp.full((_L,), l, jnp.int32)])
                pv16 = jnp.exp(tot)
                live = jnp.where(
                    jnp.logical_or(c < n_main, g >= live_from_group), 1.0, 0.0)
                pv16 = pv16 * live
                p_v[pl.ds(g * _L, _L)] = pv16
                for q in range(_L):
                    k = g * _L + q
                    ps = plsc.load_gather(p_v, [jnp.full((_L,), k, jnp.int32)])
                    for j in range(_D // _L):
                        xl_v[k, pl.ds(j * _L, _L)] = xl_v[k, pl.ds(j * _L, _L)] * ps
                return carry

            lax.fori_loop(0, _CH // _L, group, 0)

        S0, S1, S2 = sets

        issue(0, S0)
        issue(1, S1)
        wait_g(S0); compute(0, S0); issue(2, S2); scatter(S0)
        wait_g(S1); compute(1, S1); wait_s(S0); issue(3, S0); scatter(S1)
        wait_g(S2); compute(2, S2); wait_s(S1); issue(4, S1); scatter(S2)
        wait_g(S0); compute(3, S0); wait_s(S2); issue(5, S2); scatter(S0)

        def step(c, cur, prv):
            wait_g(cur)
            compute(c, cur)
            wait_s(prv)

            @pl.when(c + 2 < n_chunks)
            def _():
                issue(c + 2, prv)

            scatter(cur)

        def tri(i, carry):
            c = 4 + 3 * i
            step(c, S1, S0)
            step(c + 1, S2, S1)
            step(c + 2, S0, S2)
            return carry

        lax.fori_loop(0, (n_chunks - 4) // 3, tri, 0)
        wait_s(S0)

        plsc.subcore_barrier()

        def rdout(k, carry):
            c = sid + k * _NS

            @pl.when(c < nzc)
            def _():
                pltpu.sync_copy(acc_sh.at[pl.ds(c * _RCH, _RCH)],
                                acc_hbm.at[cid, pl.ds(c * _RCH, _RCH)])
                pltpu.sync_copy(den_sh.at[pl.ds(c * _RCH, _RCH)],
                                p1.at[pl.ds(0, _RCH)])
                pltpu.sync_copy(p1.at[pl.ds(0, _RCH)],
                                den_hbm.at[pl.ds(cid * n + c * _RCH, _RCH)])

            return carry

        lax.fori_loop(0, zk, rdout, 0)

    return sc_kernel(x_l, x_r, src, dst, att)


def kernel(x, edge_index, valid_lens, time_step_len, W_l, b_l, W_r, b_r, att, bias):
    x_l, x_r = _project(x, W_l, b_l, W_r, b_r)
    eidx = edge_index.astype(jnp.int32)
    acc, den = _sc_edge_pass(x_l, x_r, eidx[0], eidx[1], att)
    return _finalize(acc, den.reshape(_NC, x.shape[0]), bias)


# speedup vs baseline: 1.1103x; 1.1103x over previous
import functools

import jax
import jax.numpy as jnp
from jax import lax
from jax.experimental import pallas as pl
from jax.experimental.pallas import tpu as pltpu
from jax.experimental.pallas import tpu_sc as plsc

_NC = 2
_NS = 16
_NW = _NC * _NS
_L = 16

_D = 128
_CH = 80


def _proj_body(x_ref, wl_ref, bl_ref, wr_ref, br_ref, xl_ref, xr_ref):
    xb = x_ref[...]
    xl_ref[...] = jnp.dot(xb, wl_ref[...], preferred_element_type=jnp.float32) + bl_ref[...]
    xr_ref[...] = jnp.dot(xb, wr_ref[...], preferred_element_type=jnp.float32) + br_ref[...]


def _project(x, W_l, b_l, W_r, b_r):
    n, d = x.shape
    bn = 2000
    return pl.pallas_call(
        _proj_body,
        grid=(n // bn,),
        in_specs=[
            pl.BlockSpec((bn, d), lambda i: (i, 0)),
            pl.BlockSpec((d, d), lambda i: (0, 0)),
            pl.BlockSpec((1, d), lambda i: (0, 0)),
            pl.BlockSpec((d, d), lambda i: (0, 0)),
            pl.BlockSpec((1, d), lambda i: (0, 0)),
        ],
        out_specs=[
            pl.BlockSpec((bn, d), lambda i: (i, 0)),
            pl.BlockSpec((bn, d), lambda i: (i, 0)),
        ],
        out_shape=[
            jax.ShapeDtypeStruct((n, d), jnp.float32),
            jax.ShapeDtypeStruct((n, d), jnp.float32),
        ],
    )(x, W_l, b_l.reshape(1, d), W_r, b_r.reshape(1, d))


def _finalize_body(acc_ref, den_ref, bias_ref, out_ref):
    d = jnp.sum(den_ref[...], axis=0)
    a = acc_ref[0] + acc_ref[1]
    safe = jnp.where(d > 0, d, 1.0)
    out_ref[...] = a / safe[:, None] + bias_ref[...]


def _finalize(acc, den, bias):
    n = acc.shape[1]
    return pl.pallas_call(
        _finalize_body,
        out_shape=jax.ShapeDtypeStruct((n, _D), jnp.float32),
    )(acc, den, bias.reshape(1, _D))


def _sc_edge_pass(x_l, x_r, eidx_il, att):
    n = x_l.shape[0]
    e = eidx_il.shape[0] // 2
    assert e % _NW == 0
    per_tile = e // _NW
    assert per_tile % _CH == 0
    n_chunks = per_tile // _CH
    assert n_chunks % 2 == 1
    assert n % _CH == 0
    nzc = n // _CH
    zk = (nzc + _NS - 1) // _NS

    mesh = plsc.VectorSubcoreMesh(core_axis_name="c", subcore_axis_name="s",
                                  num_cores=_NC, num_subcores=_NS)

    @functools.partial(
        pl.kernel,
        out_type=[
            jax.ShapeDtypeStruct((_NC, n, _D), jnp.float32),
            jax.ShapeDtypeStruct((_NC * n,), jnp.float32),
        ],
        mesh=mesh,
        compiler_params=pltpu.CompilerParams(needs_layout_passes=False),
        scratch_types=[
            pltpu.VMEM((2 * _CH,), jnp.int32),
            pltpu.VMEM((2 * _CH,), jnp.int32),
            pltpu.VMEM((1, _CH), jnp.int32),
            pltpu.VMEM((1, _CH), jnp.int32),
            pltpu.VMEM((_CH, _D), jnp.float32),
            pltpu.VMEM((_CH, _D), jnp.float32),
            pltpu.VMEM((_CH, _D), jnp.float32),
            pltpu.VMEM((_CH, _D), jnp.float32),
            pltpu.VMEM((_CH,), jnp.float32),
            pltpu.VMEM((_CH,), jnp.float32),
            pltpu.VMEM((_D,), jnp.float32),
            pltpu.VMEM((_L * _L,), jnp.float32),
            pltpu.VMEM_SHARED((n, _D), jnp.float32),
            pltpu.VMEM_SHARED((n,), jnp.float32),
            pltpu.SemaphoreType.DMA,
            pltpu.SemaphoreType.DMA,
        ],
    )
    def sc_kernel(xl_hbm, xr_hbm, il_hbm, att_hbm, acc_hbm, den_hbm,
                  ilA, ilB, sidxA, sidxB, xlA, xrA, xlB, xrB, pA, pB,
                  att_v, ebuf, acc_sh, den_sh, semA, semB):
        cid = lax.axis_index("c")
        sid = lax.axis_index("s")
        wid = cid * _NS + sid
        base = wid * per_tile

        pltpu.sync_copy(att_hbm, att_v)

        z16 = jnp.zeros((_L,), jnp.float32)

        for q in range(_CH // _L):
            pA[pl.ds(q * _L, _L)] = z16

        def zrow(i, carry):
            xlA[i // (_D // _L), pl.ds((i % (_D // _L)) * _L, _L)] = z16
            return carry

        lax.fori_loop(0, _CH * (_D // _L), zrow, 0)

        def zacc(k, carry):
            c = sid + k * _NS

            @pl.when(c < nzc)
            def _():
                pltpu.sync_copy(xlA, acc_sh.at[pl.ds(c * _CH, _CH)])
                pltpu.sync_copy(pA, den_sh.at[pl.ds(c * _CH, _CH)])

            return carry

        lax.fori_loop(0, zk, zacc, 0)
        plsc.subcore_barrier()

        att_regs = [att_v[pl.ds(j * _L, _L)] for j in range(_D // _L)]
        lane = lax.broadcasted_iota(jnp.int32, (_L,), 0)
        lane16 = lane * _L

        def issue(ci, il_v, xl_v, xr_v, sem):
            pltpu.sync_copy(
                il_hbm.at[pl.ds(2 * base + ci * 2 * _CH, 2 * _CH)], il_v)
            pltpu.async_copy(xl_hbm.at[il_v.at[pl.ds(0, _CH)]], xl_v, sem)
            pltpu.async_copy(xr_hbm.at[il_v.at[pl.ds(_CH, _CH)]], xr_v, sem)

        def wait(il_v, xl_v, xr_v, sem):
            pltpu.make_async_copy(xl_hbm.at[il_v.at[pl.ds(0, _CH)]], xl_v,
                                  sem).wait()
            pltpu.make_async_copy(xr_hbm.at[il_v.at[pl.ds(_CH, _CH)]], xr_v,
                                  sem).wait()

        def compute_scatter(il_v, sidx, xl_v, xr_v, p_v):
            def group(g, carry):
                for q in range(_L):
                    k = g * _L + q
                    acc = jnp.zeros((_L,), jnp.float32)
                    for j in range(_D // _L):
                        a = xl_v[k, pl.ds(j * _L, _L)]
                        b = xr_v[k, pl.ds(j * _L, _L)]
                        v = a + b
                        acc = acc + jnp.maximum(v, 0.2 * v) * att_regs[j]
                    ebuf[pl.ds(q * _L, _L)] = acc
                tot = jnp.zeros((_L,), jnp.float32)
                for l in range(_L):
                    tot = tot + plsc.load_gather(ebuf, [lane16 + l])
                pv16 = jnp.exp(tot)
                p_v[pl.ds(g * _L, _L)] = pv16
                for q in range(_L):
                    k = g * _L + q
                    ps = plsc.load_gather(p_v, [jnp.full((_L,), k, jnp.int32)])
                    for j in range(_D // _L):
                        xl_v[k, pl.ds(j * _L, _L)] = xl_v[k, pl.ds(j * _L, _L)] * ps
                return carry

            lax.fori_loop(0, _CH // _L, group, 0)
            for q in range(_CH // _L):
                sidx[0, pl.ds(q * _L, _L)] = il_v[pl.ds(_CH + q * _L, _L)]
            pltpu.sync_copy(xl_v, acc_sh.at[sidx.at[0]], add=True)
            pltpu.sync_copy(p_v, den_sh.at[sidx.at[0]], add=True)

        issue(0, ilA, xlA, xrA, semA)

        def pair(i, carry):
            c0 = 2 * i
            issue(c0 + 1, ilB, xlB, xrB, semB)
            wait(ilA, xlA, xrA, semA)
            compute_scatter(ilA, sidxA, xlA, xrA, pA)
            issue(c0 + 2, ilA, xlA, xrA, semA)
            wait(ilB, xlB, xrB, semB)
            compute_scatter(ilB, sidxB, xlB, xrB, pB)
            return carry

        lax.fori_loop(0, (n_chunks - 1) // 2, pair, 0)
        wait(ilA, xlA, xrA, semA)
        compute_scatter(ilA, sidxA, xlA, xrA, pA)

        plsc.subcore_barrier()

        def rdout(k, carry):
            c = sid + k * _NS

            @pl.when(c < nzc)
            def _():
                pltpu.sync_copy(acc_sh.at[pl.ds(c * _CH, _CH)],
                                acc_hbm.at[cid, pl.ds(c * _CH, _CH)])
                pltpu.sync_copy(den_sh.at[pl.ds(c * _CH, _CH)], pB)
                pltpu.sync_copy(pB, den_hbm.at[pl.ds(cid * n + c * _CH, _CH)])

            return carry

        lax.fori_loop(0, zk, rdout, 0)

    return sc_kernel(x_l, x_r, eidx_il, att)


def kernel(x, edge_index, valid_lens, time_step_len, W_l, b_l, W_r, b_r, att, bias):
    x_l, x_r = _project(x, W_l, b_l, W_r, b_r)
    eidx = edge_index.astype(jnp.int32)
    e = eidx.shape[1]
    eidx_il = eidx.reshape(2, e // _CH, _CH).transpose(1, 0, 2).reshape(-1)
    acc, den = _sc_edge_pass(x_l, x_r, eidx_il, att)
    return _finalize(acc, den.reshape(_NC, x.shape[0]), bias)


# speedup vs baseline: 1.2710x; 1.1448x over previous
import functools

import jax
import jax.numpy as jnp
from jax import lax
from jax.experimental import pallas as pl
from jax.experimental.pallas import tpu as pltpu
from jax.experimental.pallas import tpu_sc as plsc

_NC = 2
_NS = 16
_NW = _NC * _NS
_L = 16

_D = 128
_CH = 80


def _proj_body(x_ref, wl_ref, bl_ref, wr_ref, br_ref, xl_ref, xr_ref):
    xb = x_ref[...]
    xl_ref[...] = jnp.dot(xb, wl_ref[...], preferred_element_type=jnp.float32) + bl_ref[...]
    xr_ref[...] = jnp.dot(xb, wr_ref[...], preferred_element_type=jnp.float32) + br_ref[...]


def _project(x, W_l, b_l, W_r, b_r):
    n, d = x.shape
    bn = 2000
    return pl.pallas_call(
        _proj_body,
        grid=(n // bn,),
        in_specs=[
            pl.BlockSpec((bn, d), lambda i: (i, 0)),
            pl.BlockSpec((d, d), lambda i: (0, 0)),
            pl.BlockSpec((1, d), lambda i: (0, 0)),
            pl.BlockSpec((d, d), lambda i: (0, 0)),
            pl.BlockSpec((1, d), lambda i: (0, 0)),
        ],
        out_specs=[
            pl.BlockSpec((bn, d), lambda i: (i, 0)),
            pl.BlockSpec((bn, d), lambda i: (i, 0)),
        ],
        out_shape=[
            jax.ShapeDtypeStruct((n, d), jnp.float32),
            jax.ShapeDtypeStruct((n, d), jnp.float32),
        ],
    )(x, W_l, b_l.reshape(1, d), W_r, b_r.reshape(1, d))


def _finalize_body(acc_ref, den_ref, bias_ref, out_ref):
    d = jnp.sum(den_ref[...], axis=0)
    a = acc_ref[0] + acc_ref[1]
    safe = jnp.where(d > 0, d, 1.0)
    out_ref[...] = a / safe[:, None] + bias_ref[...]


def _finalize(acc, den, bias):
    n = acc.shape[1]
    return pl.pallas_call(
        _finalize_body,
        out_shape=jax.ShapeDtypeStruct((n, _D), jnp.float32),
    )(acc, den, bias.reshape(1, _D))


def _sc_edge_pass(x_l, x_r, eidx_il, att):
    n = x_l.shape[0]
    e = eidx_il.shape[0] // 2
    assert e % _NW == 0
    per_tile = e // _NW
    assert per_tile % _CH == 0
    n_chunks = per_tile // _CH
    assert n_chunks >= 5 and (n_chunks - 1) % 4 == 0
    assert n % _CH == 0
    nzc = n // _CH
    zk = (nzc + _NS - 1) // _NS

    mesh = plsc.VectorSubcoreMesh(core_axis_name="c", subcore_axis_name="s",
                                  num_cores=_NC, num_subcores=_NS)

    @functools.partial(
        pl.kernel,
        out_type=[
            jax.ShapeDtypeStruct((_NC, n, _D), jnp.float32),
            jax.ShapeDtypeStruct((_NC * n,), jnp.float32),
        ],
        mesh=mesh,
        compiler_params=pltpu.CompilerParams(needs_layout_passes=False),
        scratch_types=[
            pltpu.VMEM((2 * _CH,), jnp.int32),
            pltpu.VMEM((2 * _CH,), jnp.int32),
            pltpu.VMEM((2 * _CH,), jnp.int32),
            pltpu.VMEM((2 * _CH,), jnp.int32),
            pltpu.VMEM((1, _CH), jnp.int32),
            pltpu.VMEM((1, _CH), jnp.int32),
            pltpu.VMEM((_CH, _D), jnp.float32),
            pltpu.VMEM((_CH, _D), jnp.float32),
            pltpu.VMEM((_CH, _D), jnp.float32),
            pltpu.VMEM((_CH, _D), jnp.float32),
            pltpu.VMEM((_CH,), jnp.float32),
            pltpu.VMEM((_CH,), jnp.float32),
            pltpu.VMEM((_D,), jnp.float32),
            pltpu.VMEM((_L * _L,), jnp.float32),
            pltpu.VMEM_SHARED((n, _D), jnp.float32),
            pltpu.VMEM_SHARED((n,), jnp.float32),
            pltpu.SemaphoreType.DMA,
            pltpu.SemaphoreType.DMA,
            pltpu.SemaphoreType.DMA,
            pltpu.SemaphoreType.DMA,
            pltpu.SemaphoreType.DMA,
            pltpu.SemaphoreType.DMA,
            pltpu.SemaphoreType.DMA,
        ],
    )
    def sc_kernel(xl_hbm, xr_hbm, il_hbm, att_hbm, acc_hbm, den_hbm,
                  il0, il1, il2, il3, sidxA, sidxB, xlA, xrA, xlB, xrB, pA, pB,
                  att_v, ebuf, acc_sh, den_sh, semA, semB,
                  isem0, isem1, isem2, isem3, ssem):
        cid = lax.axis_index("c")
        sid = lax.axis_index("s")
        wid = cid * _NS + sid
        base = wid * per_tile

        pltpu.sync_copy(att_hbm, att_v)

        z16 = jnp.zeros((_L,), jnp.float32)

        for q in range(_CH // _L):
            pA[pl.ds(q * _L, _L)] = z16

        def zrow(i, carry):
            xlA[i // (_D // _L), pl.ds((i % (_D // _L)) * _L, _L)] = z16
            return carry

        lax.fori_loop(0, _CH * (_D // _L), zrow, 0)

        def zacc(k, carry):
            c = sid + k * _NS

            @pl.when(c < nzc)
            def _():
                pltpu.sync_copy(xlA, acc_sh.at[pl.ds(c * _CH, _CH)])
                pltpu.sync_copy(pA, den_sh.at[pl.ds(c * _CH, _CH)])

            return carry

        lax.fori_loop(0, zk, zacc, 0)
        plsc.subcore_barrier()

        att_regs = [att_v[pl.ds(j * _L, _L)] for j in range(_D // _L)]
        lane = lax.broadcasted_iota(jnp.int32, (_L,), 0)
        lane16 = lane * _L

        ils = [(il0, isem0), (il1, isem1), (il2, isem2), (il3, isem3)]

        def idx_load(ci, I):
            il_v, isem = I
            pltpu.async_copy(
                il_hbm.at[pl.ds(2 * base + ci * 2 * _CH, 2 * _CH)], il_v, isem)

        def idx_wait(I):
            il_v, isem = I
            pltpu.make_async_copy(
                il_hbm.at[pl.ds(2 * base, 2 * _CH)], il_v, isem).wait()

        def gathers(I, xl_v, xr_v, sem):
            il_v, _ = I
            pltpu.async_copy(xl_hbm.at[il_v.at[pl.ds(0, _CH)]], xl_v, sem)
            pltpu.async_copy(xr_hbm.at[il_v.at[pl.ds(_CH, _CH)]], xr_v, sem)

        def wait_g(I, xl_v, xr_v, sem):
            il_v, _ = I
            pltpu.make_async_copy(xl_hbm.at[il_v.at[pl.ds(0, _CH)]], xl_v,
                                  sem).wait()
            pltpu.make_async_copy(xr_hbm.at[il_v.at[pl.ds(_CH, _CH)]], xr_v,
                                  sem).wait()

        def compute_scatter(I, sidx, xl_v, xr_v, p_v):
            il_v, _ = I
            def group(g, carry):
                for q in range(_L):
                    k = g * _L + q
                    acc = jnp.zeros((_L,), jnp.float32)
                    for j in range(_D // _L):
                        a = xl_v[k, pl.ds(j * _L, _L)]
                        b = xr_v[k, pl.ds(j * _L, _L)]
                        v = a + b
                        acc = acc + jnp.maximum(v, 0.2 * v) * att_regs[j]
                    ebuf[pl.ds(q * _L, _L)] = acc
                tot = jnp.zeros((_L,), jnp.float32)
                for l in range(_L):
                    tot = tot + plsc.load_gather(ebuf, [lane16 + l])
                pv16 = jnp.exp(tot)
                p_v[pl.ds(g * _L, _L)] = pv16
                for q in range(_L):
                    k = g * _L + q
                    ps = plsc.load_gather(p_v, [jnp.full((_L,), k, jnp.int32)])
                    for j in range(_D // _L):
                        xl_v[k, pl.ds(j * _L, _L)] = xl_v[k, pl.ds(j * _L, _L)] * ps
                return carry

            lax.fori_loop(0, _CH // _L, group, 0)
            for q in range(_CH // _L):
                sidx[0, pl.ds(q * _L, _L)] = il_v[pl.ds(_CH + q * _L, _L)]
            d1 = pltpu.async_copy(xl_v, acc_sh.at[sidx.at[0]], ssem, add=True)
            d2 = pltpu.async_copy(p_v, den_sh.at[sidx.at[0]], ssem, add=True)
            d1.wait()
            d2.wait()

        SA = (sidxA, xlA, xrA, pA, semA)
        SB = (sidxB, xlB, xrB, pB, semB)

        def chunk_step(c, I_cur, I_nxt, I_pre, S_cur, S_nxt):
            sidx, xl_v, xr_v, p_v, sem = S_cur
            _, xl_n, xr_n, _, sem_n = S_nxt
            wait_g(I_cur, xl_v, xr_v, sem)

            @pl.when(c + 1 < n_chunks)
            def _():
                idx_wait(I_nxt)
                gathers(I_nxt, xl_n, xr_n, sem_n)

            compute_scatter(I_cur, sidx, xl_v, xr_v, p_v)

            @pl.when(c + 3 < n_chunks)
            def _():
                idx_load(c + 3, I_pre)

        IL0, IL1, IL2, IL3 = ils
        idx_load(0, IL0)
        idx_load(1, IL1)
        idx_load(2, IL2)
        idx_wait(IL0)
        gathers(IL0, xlA, xrA, semA)
        chunk_step(0, IL0, IL1, IL3, SA, SB)

        def quad(i, carry):
            c = 1 + 4 * i
            chunk_step(c, IL1, IL2, IL0, SB, SA)
            chunk_step(c + 1, IL2, IL3, IL1, SA, SB)
            chunk_step(c + 2, IL3, IL0, IL2, SB, SA)
            chunk_step(c + 3, IL0, IL1, IL3, SA, SB)
            return carry

        lax.fori_loop(0, (n_chunks - 1) // 4, quad, 0)

        plsc.subcore_barrier()

        def rdout(k, carry):
            c = sid + k * _NS

            @pl.when(c < nzc)
            def _():
                pltpu.sync_copy(acc_sh.at[pl.ds(c * _CH, _CH)],
                                acc_hbm.at[cid, pl.ds(c * _CH, _CH)])
                pltpu.sync_copy(den_sh.at[pl.ds(c * _CH, _CH)], pB)
                pltpu.sync_copy(pB, den_hbm.at[pl.ds(cid * n + c * _CH, _CH)])

            return carry

        lax.fori_loop(0, zk, rdout, 0)

    return sc_kernel(x_l, x_r, eidx_il, att)


def kernel(x, edge_index, valid_lens, time_step_len, W_l, b_l, W_r, b_r, att, bias):
    x_l, x_r = _project(x, W_l, b_l, W_r, b_r)
    eidx = edge_index.astype(jnp.int32)
    e = eidx.shape[1]
    eidx_il = eidx.reshape(2, e // _CH, _CH).transpose(1, 0, 2).reshape(-1)
    acc, den = _sc_edge_pass(x_l, x_r, eidx_il, att)
    return _finalize(acc, den.reshape(_NC, x.shape[0]), bias)
